# baseline (speedup 1.0000x reference)
import functools

import jax
import jax.numpy as jnp
from jax import lax
from jax.experimental import pallas as pl
from jax.experimental.pallas import tpu as pltpu
from jax.experimental.pallas import tpu_sc as plsc

NC = 2
NS = 16
NW = NC * NS
BLK = 128
BLK_LOG = 7
D = 128


def _segsum(table, src_idx, dst_idx, n_seg, n_chunks, seg_chunk):
    d = table.shape[1]
    E = src_idx.shape[0]
    E_pad = -(-E // (NW * 16)) * (NW * 16)
    if E_pad != E:
        pad_s = jnp.zeros((E_pad - E,), jnp.int32)
        pad_d = jnp.full((E_pad - E,), jnp.int32(1 << 30))
        src_idx = jnp.concatenate([src_idx, pad_s])
        dst_idx = jnp.concatenate([dst_idx, pad_d])
    E_t = E_pad // NW
    blk = 64
    blk_log = 6
    NBLK = -(-E_t // blk)
    assert seg_chunk % 128 == 0
    rpt = seg_chunk // NS
    DUMP = seg_chunk

    mesh = plsc.VectorSubcoreMesh(core_axis_name="c", subcore_axis_name="s")

    @functools.partial(
        pl.kernel,
        out_type=jax.ShapeDtypeStruct((NC, n_chunks * seg_chunk, d),
                                      jnp.float32),
        mesh=mesh,
        compiler_params=pltpu.CompilerParams(needs_layout_passes=False),
        scratch_types=[
            pltpu.VMEM((E_t,), jnp.int32),
            pltpu.VMEM((E_t,), jnp.int32),
            pltpu.VMEM((NBLK, blk), jnp.int32),
            pltpu.VMEM((NBLK, blk), jnp.int32),
            pltpu.VMEM((3 * blk, d), jnp.float32),
            pltpu.VMEM_SHARED((seg_chunk + 8, d), jnp.float32),
            pltpu.SemaphoreType.DMA,
            pltpu.SemaphoreType.DMA,
            pltpu.SemaphoreType.DMA,
            pltpu.SemaphoreType.DMA,
        ],
    )
    def kern(table_h, src_h, dst_h, out_h, sv, dv, csrc, cdst, ring,
             acc, sem0, sem1, sem2, zsem):
        sems = (sem0, sem1, sem2)
        cid = lax.axis_index("c")
        tid = lax.axis_index("s")
        base_e = (tid * NC + cid) * E_t
        pltpu.sync_copy(src_h.at[pl.ds(base_e, E_t)], sv)
        pltpu.sync_copy(dst_h.at[pl.ds(base_e, E_t)], dv)

        z16 = jnp.zeros((16,), jnp.float32)
        zeros_i = jnp.zeros((16,), jnp.int32)
        dump_v = jnp.full((16,), DUMP, jnp.int32)
        iota16 = lax.iota(jnp.int32, 16)

        for chunk in range(n_chunks):
            lo = chunk * seg_chunk
            hi = lo + seg_chunk

            my0 = tid * rpt
            nfull = rpt // blk
            rem = rpt % blk

            def zfill(i, _):
                r = i // (d // 16)
                c = (i % (d // 16)) * 16
                ring[r, pl.ds(c, 16)] = z16
                return 0

            plsc.parallel_loop(0, blk * (d // 16), carry=jnp.int32(0))(zfill)

            for i in range(nfull):
                pltpu.async_copy(ring.at[pl.ds(0, blk)],
                                 acc.at[pl.ds(my0 + i * blk, blk)], zsem)
            if rem:
                pltpu.async_copy(ring.at[pl.ds(0, rem)],
                                 acc.at[pl.ds(my0 + nfull * blk, rem)], zsem)

            def scan_body(i, off):
                s16 = sv[pl.ds(i * 16, 16)]
                d16 = dv[pl.ds(i * 16, 16)]
                m = (d16 >= lo) & (d16 < hi)
                n = plsc.all_reduce_population_count(m)
                cum = plsc.cumsum(jnp.where(m, 1, 0))
                pos = jnp.maximum(off + cum - 1, 0)
                pr = jnp.right_shift(pos, blk_log)
                pc = jnp.bitwise_and(pos, blk - 1)
                plsc.store_scatter(csrc, [pr, pc], s16, mask=m)
                plsc.store_scatter(cdst, [pr, pc], d16 - lo, mask=m)
                return off + n[0]

            off = plsc.parallel_loop(0, E_t // 16, carry=jnp.int32(0),
                                     unroll=8)(scan_body)

            nblk = (off + blk - 1) // blk
            end = nblk * blk

            def tail_body(j, _):
                p = off + j * 16 + iota16
                m2 = p < end
                pr = jnp.right_shift(p, blk_log)
                pc = jnp.bitwise_and(p, blk - 1)
                plsc.store_scatter(csrc, [pr, pc], zeros_i, mask=m2)
                plsc.store_scatter(cdst, [pr, pc], dump_v, mask=m2)
                return jnp.int32(0)

            lax.fori_loop(0, blk // 16, tail_body, jnp.int32(0))
            for i in range(nfull):
                pltpu.make_async_copy(
                    ring.at[pl.ds(0, blk)],
                    acc.at[pl.ds(my0 + i * blk, blk)], zsem).wait()
            if rem:
                pltpu.make_async_copy(
                    ring.at[pl.ds(0, rem)],
                    acc.at[pl.ds(my0 + nfull * blk, rem)], zsem).wait()
            plsc.subcore_barrier()

            @pl.when(nblk > 0)
            def _():
                pltpu.async_copy(table_h.at[csrc.at[0]],
                                 ring.at[pl.ds(0, blk)], sems[0])

            @pl.when(nblk > 1)
            def _():
                pltpu.async_copy(table_h.at[csrc.at[1]],
                                 ring.at[pl.ds(blk, blk)], sems[1])

            def gs_group(g, _):
                for j in range(3):
                    bb = g * 3 + j

                    @pl.when(bb < nblk)
                    def _():
                        @pl.when(bb + 2 < nblk)
                        def _():
                            pltpu.async_copy(
                                table_h.at[csrc.at[bb + 2]],
                                ring.at[pl.ds(((j + 2) % 3) * blk, blk)],
                                sems[(j + 2) % 3])

                        pltpu.make_async_copy(table_h.at[csrc.at[bb]],
                                              ring.at[pl.ds(j * blk, blk)],
                                              sems[j]).wait()
                        pltpu.sync_copy(ring.at[pl.ds(j * blk, blk)],
                                        acc.at[cdst.at[bb]], add=True)
                return jnp.int32(0)

            lax.fori_loop(0, (nblk + 2) // 3, gs_group, jnp.int32(0))
            plsc.subcore_barrier()

            pltpu.sync_copy(acc.at[pl.ds(my0, rpt)],
                            out_h.at[cid, pl.ds(lo + my0, rpt)])

    out = kern(table, src_idx, dst_idx)
    return out[0, :n_seg], out[1, :n_seg]


def _gather_rows(table, idx):
    d = table.shape[1]
    B_tot = idx.shape[0]
    assert B_tot % (NW * BLK) == 0
    b_per_w = B_tot // NW
    nblk = b_per_w // BLK
    mesh = plsc.VectorSubcoreMesh(core_axis_name="c", subcore_axis_name="s")

    @functools.partial(
        pl.kernel,
        out_type=jax.ShapeDtypeStruct((B_tot, d), jnp.float32),
        mesh=mesh,
        compiler_params=pltpu.CompilerParams(needs_layout_passes=False),
        scratch_types=[
            pltpu.VMEM((b_per_w,), jnp.int32),
            pltpu.VMEM((3, BLK, d), jnp.float32),
            pltpu.SemaphoreType.DMA,
            pltpu.SemaphoreType.DMA,
            pltpu.SemaphoreType.DMA,
        ],
    )
    def kern(table_h, idx_h, out_h, idxv, ring, sem0, sem1, sem2):
        sems = (sem0, sem1, sem2)
        cid = lax.axis_index("c")
        tid = lax.axis_index("s")
        wid = tid * NC + cid
        base = wid * b_per_w
        pltpu.sync_copy(idx_h.at[pl.ds(base, b_per_w)], idxv)

        def start(b, slot):
            pltpu.async_copy(table_h.at[idxv.at[pl.ds(b * BLK, BLK)]],
                             ring.at[slot], sems[slot])

        if nblk > 0:
            start(0, 0)
        if nblk > 1:
            start(1, 1)

        def group(g, _):
            for j in range(3):
                bb = g * 3 + j

                @pl.when(bb < nblk)
                def _():
                    @pl.when(bb + 2 < nblk)
                    def _():
                        pltpu.async_copy(
                            table_h.at[idxv.at[pl.ds((bb + 2) * BLK, BLK)]],
                            ring.at[(j + 2) % 3], sems[(j + 2) % 3])

                    pltpu.make_async_copy(
                        table_h.at[idxv.at[pl.ds(bb * BLK, BLK)]],
                        ring.at[j], sems[j]).wait()
                    pltpu.sync_copy(ring.at[j],
                                    out_h.at[pl.ds(base + bb * BLK, BLK)])
            return jnp.int32(0)

        lax.fori_loop(0, (nblk + 2) // 3, group, jnp.int32(0))

    return kern(table, idx)


def _mm(x, W, adds=(), act=None, rb=512):
    N, K = x.shape
    Do = W.shape[1]
    nb = -(-N // rb)
    n_in = 1 + len(adds)

    def body(*refs):
        add_refs = refs[1:n_in]
        w_ref = refs[n_in]
        o_ref = refs[n_in + 1]
        accv = refs[0][...]
        for a in add_refs:
            accv = accv + a[...]
        y = jnp.dot(accv, w_ref[...], preferred_element_type=jnp.float32)
        if act is not None:
            y = act(y)
        o_ref[...] = y

    return pl.pallas_call(
        body,
        grid=(nb,),
        in_specs=[pl.BlockSpec((rb, K), lambda i: (i, 0))] * n_in
        + [pl.BlockSpec((K, Do), lambda i: (0, 0))],
        out_specs=pl.BlockSpec((rb, Do), lambda i: (i, 0)),
        out_shape=jax.ShapeDtypeStruct((N, Do), jnp.float32),
    )(x, *adds, W)


def _layer_attn(H0, H1, H2, Wa, va, rb=512):
    N = H0.shape[0]
    va2 = va.reshape(D, 1)

    def body(h0, h1, h2, wa, vr, o_ref):
        w = wa[...]
        v = vr[...]
        hs = [h0[...], h1[...], h2[...]]
        ss = [jnp.dot(jnp.tanh(jnp.dot(h, w, preferred_element_type=jnp.float32)),
                      v, preferred_element_type=jnp.float32) for h in hs]
        m = jnp.maximum(jnp.maximum(ss[0], ss[1]), ss[2])
        es = [jnp.exp(s - m) for s in ss]
        z = es[0] + es[1] + es[2]
        o_ref[...] = (es[0] * hs[0] + es[1] * hs[1] + es[2] * hs[2]) / z

    return pl.pallas_call(
        body,
        grid=(-(-N // rb),),
        in_specs=[pl.BlockSpec((rb, D), lambda i: (i, 0))] * 3
        + [pl.BlockSpec((D, D), lambda i: (0, 0)),
           pl.BlockSpec((D, 1), lambda i: (0, 0))],
        out_specs=pl.BlockSpec((rb, D), lambda i: (i, 0)),
        out_shape=jax.ShapeDtypeStruct((N, D), jnp.float32),
    )(H0, H1, H2, Wa, va2)


def _ins_sum_mm(g4, W_agg, BI, rb=512):
    nb = BI // rb
    specs = [
        pl.BlockSpec((rb, D), functools.partial(lambda k, i: (i + k * nb, 0), k))
        for k in range(4)
    ]

    def body(g0, g1, g2, g3, w, o_ref):
        s = g0[...] + g1[...] + g2[...] + g3[...]
        o_ref[...] = jax.nn.relu(
            jnp.dot(s, w[...], preferred_element_type=jnp.float32))

    return pl.pallas_call(
        body,
        grid=(nb,),
        in_specs=specs + [pl.BlockSpec((D, D), lambda i: (0, 0))],
        out_specs=pl.BlockSpec((rb, D), lambda i: (i, 0)),
        out_shape=jax.ShapeDtypeStruct((BI, D), jnp.float32),
    )(g4, g4, g4, g4, W_agg)


def _decoder(gx, W_dec1, W_dec2, Bn, rb=512):
    K = gx.shape[1]
    nb = Bn // rb

    def body(xs, xd, w1, w2, o_ref):
        h = jax.nn.relu(jnp.dot(xs[...] * xd[...], w1[...],
                                preferred_element_type=jnp.float32))
        o_ref[...] = jnp.dot(h, w2[...], preferred_element_type=jnp.float32)

    return pl.pallas_call(
        body,
        grid=(nb,),
        in_specs=[
            pl.BlockSpec((rb, K), lambda i: (i, 0)),
            pl.BlockSpec((rb, K), lambda i: (i + nb, 0)),
            pl.BlockSpec((K, 64), lambda i: (0, 0)),
            pl.BlockSpec((64, 1), lambda i: (0, 0)),
        ],
        out_specs=pl.BlockSpec((rb, 1), lambda i: (i, 0)),
        out_shape=jax.ShapeDtypeStruct((Bn, 1), jnp.float32),
    )(gx, gx, W_dec1, W_dec2)


def _mil_tail(ins, z, V_mil, w_mil, W_bag1, W_bag2, W_ins, Bn, I, bb=128):
    nb = Bn // bb

    def body(ins_ref, z_ref, v_ref, wm_ref, wb1, wb2, wi, o_ref):
        x = ins_ref[...]
        t = jnp.tanh(jnp.dot(x, v_ref[...], preferred_element_type=jnp.float32))
        t3 = t.reshape(bb, I, D)
        wm = wm_ref[...]
        s = jnp.sum(t3 * wm[None, :, :], axis=2)
        mx = jnp.max(s, axis=1, keepdims=True)
        e = jnp.exp(s - mx)
        attn = e / jnp.sum(e, axis=1, keepdims=True)
        x3 = x.reshape(bb, I, D)
        bag = jnp.sum(attn[:, :, None] * x3, axis=1)
        pred_bag = jnp.dot(
            jax.nn.relu(jnp.dot(bag, wb1[...], preferred_element_type=jnp.float32)),
            wb2[...], preferred_element_type=jnp.float32)
        iot = lax.broadcasted_iota(jnp.int32, (bb, I), 1)
        a = attn
        sel = jnp.zeros((bb, I), jnp.float32)
        for _ in range(3):
            m1 = jnp.max(a, axis=1, keepdims=True)
            cand = jnp.where(a == m1, iot, I)
            pick_idx = jnp.min(cand, axis=1, keepdims=True)
            pick = (iot == pick_idx)
            sel = sel + pick.astype(jnp.float32)
            a = jnp.where(pick, -jnp.inf, a)
        gm = jnp.sum(sel[:, :, None] * x3, axis=1) / 3.0
        pred_ins = jnp.dot(gm, wi[...], preferred_element_type=jnp.float32)
        o_ref[...] = (pred_ins + pred_bag + z_ref[...]) / 3.0

    return pl.pallas_call(
        body,
        grid=(nb,),
        in_specs=[
            pl.BlockSpec((bb * I, D), lambda i: (i, 0)),
            pl.BlockSpec((bb, 1), lambda i: (i, 0)),
            pl.BlockSpec((D, D), lambda i: (0, 0)),
            pl.BlockSpec((1, D), lambda i: (0, 0)),
            pl.BlockSpec((D, D), lambda i: (0, 0)),
            pl.BlockSpec((D, 1), lambda i: (0, 0)),
            pl.BlockSpec((D, 1), lambda i: (0, 0)),
        ],
        out_specs=pl.BlockSpec((bb, 1), lambda i: (i, 0)),
        out_shape=jax.ShapeDtypeStruct((Bn, 1), jnp.float32),
    )(ins, z, V_mil, w_mil.reshape(1, D), W_bag1, W_bag2, W_ins)


def kernel(feat_miRNA, feat_drug, feat_gene, W_lin_m, W_lin_d, W_lin_g,
           W_e0_m, W_e0_d, W_e0_g, W_e1_m, W_e1_d, W_e1_g,
           Wa_m, va_m, Wa_d, va_d, Wa_g, va_g, W_agg, V_mil, w_mil,
           W_bag1, W_bag2, W_gcn1, W_gcn2, W_dec1, W_dec2, W_ins,
           edge_md, edge_mg, edge_dg, mp_ins):
    N_M, N_D, N_G = feat_miRNA.shape[0], feat_drug.shape[0], feat_gene.shape[0]
    Bn, I = mp_ins.shape[0], mp_ins.shape[1]
    relu = jax.nn.relu

    hm = _mm(feat_miRNA, W_lin_m, act=relu, rb=128)
    hd = _mm(feat_drug, W_lin_d, act=relu)
    hg = _mm(feat_gene, W_lin_g, act=relu)

    md0, md1 = edge_md[0], edge_md[1]
    mg0, mg1 = edge_mg[0], edge_mg[1]
    dg0, dg1 = edge_dg[0], edge_dg[1]

    off_hd = N_M
    off_hg = N_M + N_D
    srcM = jnp.concatenate([md1 + off_hd, mg1 + off_hg])
    dstM = jnp.concatenate([md0, mg0])
    srcD = jnp.concatenate([md0, dg1 + off_hg])
    dstD = jnp.concatenate([md1, dg0])

    Hm, Hd, Hg = [hm], [hd], [hg]
    for Wm, Wd, Wg in ((W_e0_m, W_e0_d, W_e0_g), (W_e1_m, W_e1_d, W_e1_g)):
        bigtab = jnp.concatenate([hm, hd, hg], axis=0)
        m_p = _segsum(bigtab, srcM, dstM, N_M, 1, 640)
        d_p = _segsum(bigtab, srcD, dstD, N_D, 3, 2688)
        g_m = _segsum(hm, mg0, mg1, N_G, 9, 5632)
        g_d = _segsum(hd, dg0, dg1, N_G, 9, 5632)
        hm = _mm(hm, Wm, adds=m_p, act=relu, rb=128)
        hd = _mm(hd, Wd, adds=d_p, act=relu)
        hg = _mm(hg, Wg, adds=g_m + g_d, act=relu)
        Hm.append(hm)
        Hd.append(hd)
        Hg.append(hg)

    hm = _layer_attn(Hm[0], Hm[1], Hm[2], Wa_m, va_m, rb=128)
    hd = _layer_attn(Hd[0], Hd[1], Hd[2], Wa_d, va_d)
    hg = _layer_attn(Hg[0], Hg[1], Hg[2], Wa_g, va_g)

    big_table = jnp.concatenate([hm, hg, hd], axis=0)
    off_g = N_M
    off_d = N_M + N_G
    idx_all = jnp.concatenate([
        mp_ins[:, :, 0].reshape(-1),
        mp_ins[:, :, 1].reshape(-1) + off_g,
        mp_ins[:, :, 2].reshape(-1) + off_g,
        mp_ins[:, :, 3].reshape(-1) + off_d,
    ])
    g4 = _gather_rows(big_table, idx_all)
    BI = Bn * I
    ins = _ins_sum_mm(g4, W_agg, BI)

    combined = jnp.concatenate([hm, hd], axis=0)
    nc = N_M + N_D
    src = mp_ins[:, 0, 0]
    dst = mp_ins[:, 0, 3] + N_M
    x = combined
    src2 = jnp.concatenate([src, dst])
    dst2 = jnp.concatenate([dst, src])
    for W in (W_gcn1, W_gcn2):
        agg = _segsum(x, src2, dst2, nc, 2, 4352)
        x = _mm(x, W, adds=agg, act=relu)

    gx = _gather_rows(x, jnp.concatenate([src, dst]))
    z = _decoder(gx, W_dec1, W_dec2, Bn)

    return _mil_tail(ins, z, V_mil, w_mil, W_bag1, W_bag2, W_ins, Bn, I)


# speedup vs baseline: 1.0512x; 1.0512x over previous
import functools

import jax
import jax.numpy as jnp
from jax import lax
from jax.experimental import pallas as pl
from jax.experimental.pallas import tpu as pltpu
from jax.experimental.pallas import tpu_sc as plsc

NC = 2
NS = 16
NW = NC * NS
BLK = 128
BLK_LOG = 7
D = 128


def _segsum(table, src_idx, dst_idx, n_seg, n_chunks, seg_chunk):
    d = table.shape[1]
    E = src_idx.shape[0]
    E_pad = -(-E // (NW * 16)) * (NW * 16)
    if E_pad != E:
        pad_s = jnp.zeros((E_pad - E,), jnp.int32)
        pad_d = jnp.full((E_pad - E,), jnp.int32(1 << 30))
        src_idx = jnp.concatenate([src_idx, pad_s])
        dst_idx = jnp.concatenate([dst_idx, pad_d])
    E_t = E_pad // NW
    blk = 64
    blk_log = 6
    NBLK = -(-E_t // blk)
    assert seg_chunk % 128 == 0
    rpt = seg_chunk // NS
    DUMP = seg_chunk

    mesh = plsc.VectorSubcoreMesh(core_axis_name="c", subcore_axis_name="s")

    @functools.partial(
        pl.kernel,
        out_type=jax.ShapeDtypeStruct((NC, n_chunks * seg_chunk, d),
                                      jnp.float32),
        mesh=mesh,
        compiler_params=pltpu.CompilerParams(needs_layout_passes=False),
        scratch_types=[
            pltpu.VMEM((E_t,), jnp.int32),
            pltpu.VMEM((E_t,), jnp.int32),
            pltpu.VMEM((NBLK, blk), jnp.int32),
            pltpu.VMEM((NBLK, blk), jnp.int32),
            pltpu.VMEM((3 * blk, d), jnp.float32),
            pltpu.VMEM_SHARED((seg_chunk + 8, d), jnp.float32),
            pltpu.SemaphoreType.DMA,
            pltpu.SemaphoreType.DMA,
            pltpu.SemaphoreType.DMA,
            pltpu.SemaphoreType.DMA,
        ],
    )
    def kern(table_h, src_h, dst_h, out_h, sv, dv, csrc, cdst, ring,
             acc, sem0, sem1, sem2, zsem):
        sems = (sem0, sem1, sem2)
        cid = lax.axis_index("c")
        tid = lax.axis_index("s")
        base_e = (tid * NC + cid) * E_t
        pltpu.sync_copy(src_h.at[pl.ds(base_e, E_t)], sv)
        pltpu.sync_copy(dst_h.at[pl.ds(base_e, E_t)], dv)

        z16 = jnp.zeros((16,), jnp.float32)
        zeros_i = jnp.zeros((16,), jnp.int32)
        dump_v = jnp.full((16,), DUMP, jnp.int32)
        iota16 = lax.iota(jnp.int32, 16)

        for chunk in range(n_chunks):
            lo = chunk * seg_chunk
            hi = lo + seg_chunk

            my0 = tid * rpt
            nfull = rpt // blk
            rem = rpt % blk

            def zfill(i, _):
                r = i // (d // 16)
                c = (i % (d // 16)) * 16
                ring[r, pl.ds(c, 16)] = z16
                return 0

            plsc.parallel_loop(0, blk * (d // 16), carry=jnp.int32(0))(zfill)

            for i in range(nfull):
                pltpu.async_copy(ring.at[pl.ds(0, blk)],
                                 acc.at[pl.ds(my0 + i * blk, blk)], zsem)
            if rem:
                pltpu.async_copy(ring.at[pl.ds(0, rem)],
                                 acc.at[pl.ds(my0 + nfull * blk, rem)], zsem)

            def scan_body(i, off):
                s16 = sv[pl.ds(i * 16, 16)]
                d16 = dv[pl.ds(i * 16, 16)]
                m = (d16 >= lo) & (d16 < hi)
                n = plsc.all_reduce_population_count(m)
                cum = plsc.cumsum(jnp.where(m, 1, 0))
                pos = jnp.maximum(off + cum - 1, 0)
                pr = jnp.right_shift(pos, blk_log)
                pc = jnp.bitwise_and(pos, blk - 1)
                plsc.store_scatter(csrc, [pr, pc], s16, mask=m)
                plsc.store_scatter(cdst, [pr, pc], d16 - lo, mask=m)
                return off + n[0]

            off = plsc.parallel_loop(0, E_t // 16, carry=jnp.int32(0),
                                     unroll=8)(scan_body)

            nblk = (off + blk - 1) // blk
            end = nblk * blk

            def tail_body(j, _):
                p = off + j * 16 + iota16
                m2 = p < end
                pr = jnp.right_shift(p, blk_log)
                pc = jnp.bitwise_and(p, blk - 1)
                plsc.store_scatter(csrc, [pr, pc], zeros_i, mask=m2)
                plsc.store_scatter(cdst, [pr, pc], dump_v, mask=m2)
                return jnp.int32(0)

            lax.fori_loop(0, blk // 16, tail_body, jnp.int32(0))
            for i in range(nfull):
                pltpu.make_async_copy(
                    ring.at[pl.ds(0, blk)],
                    acc.at[pl.ds(my0 + i * blk, blk)], zsem).wait()
            if rem:
                pltpu.make_async_copy(
                    ring.at[pl.ds(0, rem)],
                    acc.at[pl.ds(my0 + nfull * blk, rem)], zsem).wait()
            plsc.subcore_barrier()

            @pl.when(nblk > 0)
            def _():
                pltpu.async_copy(table_h.at[csrc.at[0]],
                                 ring.at[pl.ds(0, blk)], sems[0])

            @pl.when(nblk > 1)
            def _():
                pltpu.async_copy(table_h.at[csrc.at[1]],
                                 ring.at[pl.ds(blk, blk)], sems[1])

            def gs_group(g, _):
                for j in range(3):
                    bb = g * 3 + j

                    @pl.when(bb < nblk)
                    def _():
                        @pl.when(bb + 2 < nblk)
                        def _():
                            pltpu.async_copy(
                                table_h.at[csrc.at[bb + 2]],
                                ring.at[pl.ds(((j + 2) % 3) * blk, blk)],
                                sems[(j + 2) % 3])

                        pltpu.make_async_copy(table_h.at[csrc.at[bb]],
                                              ring.at[pl.ds(j * blk, blk)],
                                              sems[j]).wait()
                        pltpu.sync_copy(ring.at[pl.ds(j * blk, blk)],
                                        acc.at[cdst.at[bb]], add=True)
                return jnp.int32(0)

            lax.fori_loop(0, (nblk + 2) // 3, gs_group, jnp.int32(0))
            plsc.subcore_barrier()

            pltpu.sync_copy(acc.at[pl.ds(my0, rpt)],
                            out_h.at[cid, pl.ds(lo + my0, rpt)])

    out = kern(table, src_idx, dst_idx)
    return out[0, :n_seg], out[1, :n_seg]


def _gather_rows(table, idx):
    d = table.shape[1]
    B_tot = idx.shape[0]
    assert B_tot % (NW * BLK) == 0
    b_per_w = B_tot // NW
    nblk = b_per_w // BLK
    mesh = plsc.VectorSubcoreMesh(core_axis_name="c", subcore_axis_name="s")

    @functools.partial(
        pl.kernel,
        out_type=jax.ShapeDtypeStruct((B_tot, d), jnp.float32),
        mesh=mesh,
        compiler_params=pltpu.CompilerParams(needs_layout_passes=False),
        scratch_types=[
            pltpu.VMEM((b_per_w,), jnp.int32),
            pltpu.VMEM((3, BLK, d), jnp.float32),
            pltpu.SemaphoreType.DMA,
            pltpu.SemaphoreType.DMA,
            pltpu.SemaphoreType.DMA,
        ],
    )
    def kern(table_h, idx_h, out_h, idxv, ring, sem0, sem1, sem2):
        sems = (sem0, sem1, sem2)
        cid = lax.axis_index("c")
        tid = lax.axis_index("s")
        wid = tid * NC + cid
        base = wid * b_per_w
        pltpu.sync_copy(idx_h.at[pl.ds(base, b_per_w)], idxv)

        def start(b, slot):
            pltpu.async_copy(table_h.at[idxv.at[pl.ds(b * BLK, BLK)]],
                             ring.at[slot], sems[slot])

        if nblk > 0:
            start(0, 0)
        if nblk > 1:
            start(1, 1)

        def group(g, _):
            for j in range(3):
                bb = g * 3 + j

                @pl.when(bb < nblk)
                def _():
                    @pl.when(bb + 2 < nblk)
                    def _():
                        pltpu.async_copy(
                            table_h.at[idxv.at[pl.ds((bb + 2) * BLK, BLK)]],
                            ring.at[(j + 2) % 3], sems[(j + 2) % 3])

                    pltpu.make_async_copy(
                        table_h.at[idxv.at[pl.ds(bb * BLK, BLK)]],
                        ring.at[j], sems[j]).wait()
                    pltpu.sync_copy(ring.at[j],
                                    out_h.at[pl.ds(base + bb * BLK, BLK)])
            return jnp.int32(0)

        lax.fori_loop(0, (nblk + 2) // 3, group, jnp.int32(0))

    return kern(table, idx)


def _mm(x, W, adds=(), act=None, rb=512):
    N, K = x.shape
    Do = W.shape[1]
    nb = -(-N // rb)
    n_in = 1 + len(adds)

    def body(*refs):
        add_refs = refs[1:n_in]
        w_ref = refs[n_in]
        o_ref = refs[n_in + 1]
        accv = refs[0][...]
        for a in add_refs:
            accv = accv + a[...]
        y = jnp.dot(accv, w_ref[...], preferred_element_type=jnp.float32)
        if act is not None:
            y = act(y)
        o_ref[...] = y

    return pl.pallas_call(
        body,
        grid=(nb,),
        in_specs=[pl.BlockSpec((rb, K), lambda i: (i, 0))] * n_in
        + [pl.BlockSpec((K, Do), lambda i: (0, 0))],
        out_specs=pl.BlockSpec((rb, Do), lambda i: (i, 0)),
        out_shape=jax.ShapeDtypeStruct((N, Do), jnp.float32),
    )(x, *adds, W)


def _layer_attn(H0, H1, H2, Wa, va, rb=512):
    N = H0.shape[0]
    va2 = va.reshape(D, 1)

    def body(h0, h1, h2, wa, vr, o_ref):
        w = wa[...]
        v = vr[...]
        hs = [h0[...], h1[...], h2[...]]
        ss = [jnp.dot(jnp.tanh(jnp.dot(h, w, preferred_element_type=jnp.float32)),
                      v, preferred_element_type=jnp.float32) for h in hs]
        m = jnp.maximum(jnp.maximum(ss[0], ss[1]), ss[2])
        es = [jnp.exp(s - m) for s in ss]
        z = es[0] + es[1] + es[2]
        o_ref[...] = (es[0] * hs[0] + es[1] * hs[1] + es[2] * hs[2]) / z

    return pl.pallas_call(
        body,
        grid=(-(-N // rb),),
        in_specs=[pl.BlockSpec((rb, D), lambda i: (i, 0))] * 3
        + [pl.BlockSpec((D, D), lambda i: (0, 0)),
           pl.BlockSpec((D, 1), lambda i: (0, 0))],
        out_specs=pl.BlockSpec((rb, D), lambda i: (i, 0)),
        out_shape=jax.ShapeDtypeStruct((N, D), jnp.float32),
    )(H0, H1, H2, Wa, va2)


def _ins_sum_mm(g4, W_agg, BI, rb=512):
    nb = BI // rb
    specs = [
        pl.BlockSpec((rb, D), functools.partial(lambda k, i: (i + k * nb, 0), k))
        for k in range(4)
    ]

    def body(g0, g1, g2, g3, w, o_ref):
        s = g0[...] + g1[...] + g2[...] + g3[...]
        o_ref[...] = jax.nn.relu(
            jnp.dot(s, w[...], preferred_element_type=jnp.float32))

    return pl.pallas_call(
        body,
        grid=(nb,),
        in_specs=specs + [pl.BlockSpec((D, D), lambda i: (0, 0))],
        out_specs=pl.BlockSpec((rb, D), lambda i: (i, 0)),
        out_shape=jax.ShapeDtypeStruct((BI, D), jnp.float32),
    )(g4, g4, g4, g4, W_agg)


def _decoder(gx, W_dec1, W_dec2, Bn, rb=512):
    K = gx.shape[1]
    nb = Bn // rb

    def body(xs, xd, w1, w2, o_ref):
        h = jax.nn.relu(jnp.dot(xs[...] * xd[...], w1[...],
                                preferred_element_type=jnp.float32))
        o_ref[...] = jnp.dot(h, w2[...], preferred_element_type=jnp.float32)

    return pl.pallas_call(
        body,
        grid=(nb,),
        in_specs=[
            pl.BlockSpec((rb, K), lambda i: (i, 0)),
            pl.BlockSpec((rb, K), lambda i: (i + nb, 0)),
            pl.BlockSpec((K, 64), lambda i: (0, 0)),
            pl.BlockSpec((64, 1), lambda i: (0, 0)),
        ],
        out_specs=pl.BlockSpec((rb, 1), lambda i: (i, 0)),
        out_shape=jax.ShapeDtypeStruct((Bn, 1), jnp.float32),
    )(gx, gx, W_dec1, W_dec2)


def _mil_tail(ins, z, V_mil, w_mil, W_bag1, W_bag2, W_ins, Bn, I, bb=128):
    nb = Bn // bb

    def body(ins_ref, z_ref, v_ref, wm_ref, wb1, wb2, wi, o_ref):
        x = ins_ref[...]
        t = jnp.tanh(jnp.dot(x, v_ref[...], preferred_element_type=jnp.float32))
        t3 = t.reshape(bb, I, D)
        wm = wm_ref[...]
        s = jnp.sum(t3 * wm[None, :, :], axis=2)
        mx = jnp.max(s, axis=1, keepdims=True)
        e = jnp.exp(s - mx)
        attn = e / jnp.sum(e, axis=1, keepdims=True)
        x3 = x.reshape(bb, I, D)
        bag = jnp.sum(attn[:, :, None] * x3, axis=1)
        pred_bag = jnp.dot(
            jax.nn.relu(jnp.dot(bag, wb1[...], preferred_element_type=jnp.float32)),
            wb2[...], preferred_element_type=jnp.float32)
        iot = lax.broadcasted_iota(jnp.int32, (bb, I), 1)
        a = attn
        sel = jnp.zeros((bb, I), jnp.float32)
        for _ in range(3):
            m1 = jnp.max(a, axis=1, keepdims=True)
            cand = jnp.where(a == m1, iot, I)
            pick_idx = jnp.min(cand, axis=1, keepdims=True)
            pick = (iot == pick_idx)
            sel = sel + pick.astype(jnp.float32)
            a = jnp.where(pick, -jnp.inf, a)
        gm = jnp.sum(sel[:, :, None] * x3, axis=1) / 3.0
        pred_ins = jnp.dot(gm, wi[...], preferred_element_type=jnp.float32)
        o_ref[...] = (pred_ins + pred_bag + z_ref[...]) / 3.0

    return pl.pallas_call(
        body,
        grid=(nb,),
        in_specs=[
            pl.BlockSpec((bb * I, D), lambda i: (i, 0)),
            pl.BlockSpec((bb, 1), lambda i: (i, 0)),
            pl.BlockSpec((D, D), lambda i: (0, 0)),
            pl.BlockSpec((1, D), lambda i: (0, 0)),
            pl.BlockSpec((D, D), lambda i: (0, 0)),
            pl.BlockSpec((D, 1), lambda i: (0, 0)),
            pl.BlockSpec((D, 1), lambda i: (0, 0)),
        ],
        out_specs=pl.BlockSpec((bb, 1), lambda i: (i, 0)),
        out_shape=jax.ShapeDtypeStruct((Bn, 1), jnp.float32),
    )(ins, z, V_mil, w_mil.reshape(1, D), W_bag1, W_bag2, W_ins)


def kernel(feat_miRNA, feat_drug, feat_gene, W_lin_m, W_lin_d, W_lin_g,
           W_e0_m, W_e0_d, W_e0_g, W_e1_m, W_e1_d, W_e1_g,
           Wa_m, va_m, Wa_d, va_d, Wa_g, va_g, W_agg, V_mil, w_mil,
           W_bag1, W_bag2, W_gcn1, W_gcn2, W_dec1, W_dec2, W_ins,
           edge_md, edge_mg, edge_dg, mp_ins):
    N_M, N_D, N_G = feat_miRNA.shape[0], feat_drug.shape[0], feat_gene.shape[0]
    Bn, I = mp_ins.shape[0], mp_ins.shape[1]
    relu = jax.nn.relu

    hm = _mm(feat_miRNA, W_lin_m, act=relu, rb=128)
    hd = _mm(feat_drug, W_lin_d, act=relu)
    hg = _mm(feat_gene, W_lin_g, act=relu)

    md0, md1 = edge_md[0], edge_md[1]
    mg0, mg1 = edge_mg[0], edge_mg[1]
    dg0, dg1 = edge_dg[0], edge_dg[1]

    Hm, Hd, Hg = [hm], [hd], [hg]
    for Wm, Wd, Wg in ((W_e0_m, W_e0_d, W_e0_g), (W_e1_m, W_e1_d, W_e1_g)):
        m_d = _segsum(hd, md1, md0, N_M, 1, 640)
        m_g = _segsum(hg, mg1, mg0, N_M, 1, 640)
        d_m = _segsum(hm, md0, md1, N_D, 2, 4096)
        d_g = _segsum(hg, dg1, dg0, N_D, 2, 4096)
        g_m = _segsum(hm, mg0, mg1, N_G, 9, 5632)
        g_d = _segsum(hd, dg0, dg1, N_G, 9, 5632)
        hm = _mm(hm, Wm, adds=m_d + m_g, act=relu, rb=128)
        hd = _mm(hd, Wd, adds=d_m + d_g, act=relu)
        hg = _mm(hg, Wg, adds=g_m + g_d, act=relu)
        Hm.append(hm)
        Hd.append(hd)
        Hg.append(hg)

    hm = _layer_attn(Hm[0], Hm[1], Hm[2], Wa_m, va_m, rb=128)
    hd = _layer_attn(Hd[0], Hd[1], Hd[2], Wa_d, va_d)
    hg = _layer_attn(Hg[0], Hg[1], Hg[2], Wa_g, va_g)

    big_table = jnp.concatenate([hm, hg, hd], axis=0)
    off_g = N_M
    off_d = N_M + N_G
    idx_all = jnp.concatenate([
        mp_ins[:, :, 0].reshape(-1),
        mp_ins[:, :, 1].reshape(-1) + off_g,
        mp_ins[:, :, 2].reshape(-1) + off_g,
        mp_ins[:, :, 3].reshape(-1) + off_d,
    ])
    g4 = _gather_rows(big_table, idx_all)
    BI = Bn * I
    ins = _ins_sum_mm(g4, W_agg, BI)

    combined = jnp.concatenate([hm, hd], axis=0)
    nc = N_M + N_D
    src = mp_ins[:, 0, 0]
    dst = mp_ins[:, 0, 3] + N_M
    x = combined
    src2 = jnp.concatenate([src, dst])
    dst2 = jnp.concatenate([dst, src])
    for W in (W_gcn1, W_gcn2):
        agg = _segsum(x, src2, dst2, nc, 2, 4352)
        x = _mm(x, W, adds=agg, act=relu)

    gx = _gather_rows(x, jnp.concatenate([src, dst]))
    z = _decoder(gx, W_dec1, W_dec2, Bn)

    return _mil_tail(ins, z, V_mil, w_mil, W_bag1, W_bag2, W_ins, Bn, I)
